# baseline (speedup 1.0000x reference)
import functools

import numpy as np
import jax
import jax.numpy as jnp
from jax import lax
from jax.experimental import pallas as pl
from jax.experimental.pallas import tpu as pltpu
from jax.experimental.pallas import tpu_sc as plsc

_LANES = 16
_PAT_ROWS = 8


def _embed_kernel(B, F, n_cat, V, E, n_num, n_workers, chunk):
    rows_per_w = B // n_workers
    n_sub = rows_per_w // chunk
    pat = _PAT_ROWS * n_cat
    n_vec = pat // _LANES
    lk_per_sub = chunk * n_cat
    n_g = lk_per_sub // 128
    mesh = plsc.VectorSubcoreMesh(core_axis_name="c", subcore_axis_name="s")

    @functools.partial(
        pl.kernel,
        out_type=jax.ShapeDtypeStruct((B * n_cat, E), jnp.float32),
        mesh=mesh,
        compiler_params=pltpu.CompilerParams(
            use_tc_tiling_on_sc=False, needs_layout_passes=False
        ),
        scratch_types=[
            pltpu.VMEM((chunk, F), jnp.float32),
            pltpu.VMEM((pat,), jnp.int32),
            pltpu.VMEM((pat,), jnp.int32),
            pltpu.VMEM((pat,), jnp.int32),
            pltpu.VMEM((n_g, 128), jnp.int32),
            pltpu.VMEM((lk_per_sub, E), jnp.float32),
            pltpu.SemaphoreType.DMA,
            pltpu.SemaphoreType.DMA,
        ],
    )
    def k(x_hbm, tab_hbm, brow_hbm, bcol_hbm, toff_hbm, out_hbm,
          x_v, brow_v, bcol_v, toff_v, idx_v, emb_v, gsem, ssem):
        wid = lax.axis_index("s") * 2 + lax.axis_index("c")
        base = wid * rows_per_w
        pltpu.sync_copy(brow_hbm, brow_v)
        pltpu.sync_copy(bcol_hbm, bcol_v)
        pltpu.sync_copy(toff_hbm, toff_v)

        @pl.loop(0, n_sub)
        def _sub(sub):
            rowbase = base + sub * chunk
            pltpu.sync_copy(x_hbm.at[pl.ds(rowbase, chunk), :], x_v)

            @pl.loop(0, chunk // _PAT_ROWS)
            def _g(g):
                for v in range(n_vec):
                    sl = pl.ds(v * _LANES, _LANES)
                    rows = brow_v[sl] + g * _PAT_ROWS
                    vals = plsc.load_gather(x_v, [rows, bcol_v[sl]])
                    idx = vals.astype(jnp.int32) + toff_v[sl]
                    flat = g * pat + v * _LANES
                    q = flat // 128
                    r = flat % 128
                    idx_v[q, pl.ds(r, _LANES)] = idx

            gathers = [
                pltpu.async_copy(
                    tab_hbm.at[idx_v.at[q]],
                    emb_v.at[pl.ds(q * 128, 128)],
                    gsem,
                )
                for q in range(n_g)
            ]
            for g in gathers:
                g.wait()

            pltpu.async_copy(
                emb_v,
                out_hbm.at[pl.ds(rowbase * n_cat, lk_per_sub), :],
                ssem,
            ).wait()

    return k


def kernel(x, tables):
    B, F = x.shape
    n_cat, V, E = tables.shape
    n_num = F - n_cat
    tab_flat = tables.reshape(n_cat * V, E)

    p = np.arange(_PAT_ROWS * n_cat)
    brow = (p // n_cat).astype(np.int32)
    bcol = (p % n_cat + n_num).astype(np.int32)
    toff = ((p % n_cat) * V).astype(np.int32)

    k = _embed_kernel(B, F, n_cat, V, E, n_num, n_workers=32, chunk=64)
    emb = k(x, tab_flat, jnp.asarray(brow), jnp.asarray(bcol), jnp.asarray(toff))
    return jnp.concatenate([x[:, :n_num], emb.reshape(B, n_cat * E)], axis=1)


# speedup vs baseline: 1.1393x; 1.1393x over previous
import functools

import jax
import jax.numpy as jnp
from jax import lax
from jax.experimental import pallas as pl
from jax.experimental.pallas import tpu as pltpu
from jax.experimental.pallas import tpu_sc as plsc


def _embed_kernel(B, n_cat, V, E):
    n_blk = E // 8
    n_tasks = n_cat * n_blk
    tasks_per_sc = n_tasks // 2
    half = B // 2
    qchunk = half // 4
    V128 = (V // 128) * 128
    vtail = V - V128
    plen = 12544
    pieces = [(i * plen, plen) for i in range(V128 // plen)]
    if V128 % plen:
        pieces.append(((V128 // plen) * plen, V128 % plen))
    tlen = pieces[-1][1]
    mesh = plsc.VectorSubcoreMesh(core_axis_name="c", subcore_axis_name="s")

    @functools.partial(
        pl.kernel,
        out_type=jax.ShapeDtypeStruct((n_cat * E, B), jnp.float32),
        mesh=mesh,
        compiler_params=pltpu.CompilerParams(needs_layout_passes=False),
        scratch_types=[
            pltpu.VMEM_SHARED((8, plen), jnp.float32),
            pltpu.VMEM_SHARED((8, tlen), jnp.float32),
            pltpu.VMEM_SHARED((8, B), jnp.float32),
            pltpu.VMEM((V,), jnp.float32),
            pltpu.VMEM((16, 128), jnp.int32),
            pltpu.VMEM((qchunk,), jnp.float32),
            pltpu.VMEM((8, vtail), jnp.float32),
        ],
    )
    def k(tab_hbm, cat_hbm, out_hbm, pbuf, ptail, stage,
          row_v, cat_v, out_v, tail_v):
        c = lax.axis_index("c")
        t = lax.axis_index("s")
        e = t % 8
        h = t // 8
        task0 = c * tasks_per_sc

        @pl.loop(0, tasks_per_sc)
        def _task(p):
            tid = task0 + p
            j = tid // n_blk
            sp = tid % n_blk

            for (poff, pln) in pieces:
                buf = pbuf if pln == plen else ptail
                @pl.when(t == 0)
                def _():
                    pltpu.sync_copy(
                        tab_hbm.at[j, pl.ds(sp * 8, 8), pl.ds(poff, pln)],
                        buf,
                    )
                plsc.subcore_barrier()
                pltpu.sync_copy(buf.at[e, :], row_v.at[pl.ds(poff, pln)])
                plsc.subcore_barrier()

            pltpu.sync_copy(tab_hbm.at[j, pl.ds(sp * 8, 8), pl.ds(V128, vtail)],
                            tail_v)
            for w in range(vtail // 16):
                row_v[pl.ds(V128 + w * 16, 16)] = tail_v[e, pl.ds(w * 16, 16)]

            for qq in range(4):
                pltpu.sync_copy(
                    cat_hbm.at[j, pl.ds(h * 64 + qq * 16, 16), :], cat_v
                )

                @pl.loop(0, 16)
                def _rows(a):
                    for bb in range(8):
                        ii = cat_v[a, pl.ds(bb * 16, 16)]
                        vals = plsc.load_gather(row_v, [ii])
                        out_v[pl.ds(a * 128 + bb * 16, 16)] = vals

                pltpu.sync_copy(
                    out_v, stage.at[e, pl.ds(h * half + qq * qchunk, qchunk)]
                )

            plsc.subcore_barrier()

            @pl.when(t == 15)
            def _flush():
                pltpu.sync_copy(stage, out_hbm.at[pl.ds(j * E + sp * 8, 8), :])
            plsc.subcore_barrier()

    return k


def kernel(x, tables):
    B, F = x.shape
    n_cat, V, E = tables.shape
    n_num = F - n_cat

    tab_t = jnp.transpose(tables, (0, 2, 1))
    cat_js = x[:, n_num:].astype(jnp.int32).T.reshape(n_cat, B // 128, 128)

    k = _embed_kernel(B, n_cat, V, E)
    emb_t = k(tab_t, cat_js)
    return jnp.concatenate([x[:, :n_num], emb_t.T], axis=1)


# speedup vs baseline: 1.5743x; 1.3819x over previous
import functools

import jax
import jax.numpy as jnp
from jax import lax
from jax.experimental import pallas as pl
from jax.experimental.pallas import tpu as pltpu
from jax.experimental.pallas import tpu_sc as plsc


def _embed_kernel(B, n_cat, V, E):
    n_blk = E // 8
    n_tasks = n_cat * n_blk
    tasks_per_sc = n_tasks // 2
    half = B // 2
    qchunk = half // 4
    V128 = (V // 128) * 128
    vtail = V - V128
    plen = 9728
    pieces = [(i * plen, plen) for i in range(V128 // plen)]
    if V128 % plen:
        pieces.append(((V128 // plen) * plen, V128 % plen))
    n_pc = len(pieces)
    NBUF = 3
    NQ = 4
    qlen = plen // NQ
    mesh = plsc.VectorSubcoreMesh(core_axis_name="c", subcore_axis_name="s")

    @functools.partial(
        pl.kernel,
        out_type=jax.ShapeDtypeStruct((n_cat * E, B), jnp.float32),
        mesh=mesh,
        compiler_params=pltpu.CompilerParams(needs_layout_passes=False),
        scratch_types=[
            pltpu.VMEM_SHARED((8, plen), jnp.float32),
            pltpu.VMEM_SHARED((8, plen), jnp.float32),
            pltpu.VMEM_SHARED((8, plen), jnp.float32),
            pltpu.VMEM_SHARED((8, B), jnp.float32),
            pltpu.VMEM((V,), jnp.float32),
            pltpu.VMEM((16, 128), jnp.int32),
            pltpu.VMEM((qchunk,), jnp.float32),
            pltpu.VMEM((8, vtail), jnp.float32),
        ] + [pltpu.SemaphoreType.DMA] * (3 * 4) + [
        ],
    )
    def k(tab_hbm, cat_hbm, out_hbm, pbuf0, pbuf1, pbuf2, stage,
          row_v, cat_v, out_v, tail_v, *sems):
        c = lax.axis_index("c")
        t = lax.axis_index("s")
        e = t % 8
        h = t // 8
        task0 = c * tasks_per_sc
        bufs = (pbuf0, pbuf1, pbuf2)

        def piece_copy(j, sp, i):
            poff, pln = pieces[i]
            b = i % NBUF
            if pln == plen:
                cps = []
                for q in range(NQ):
                    cps.append(pltpu.make_async_copy(
                        tab_hbm.at[j, pl.ds(sp * 8, 8),
                                   pl.ds(poff + q * qlen, qlen)],
                        bufs[b].at[:, pl.ds(q * qlen, qlen)],
                        sems[b * NQ + q],
                    ))
                return cps
            return [pltpu.make_async_copy(
                tab_hbm.at[j, pl.ds(sp * 8, 8), pl.ds(poff, pln)],
                bufs[b].at[:, pl.ds(0, pln)],
                sems[b * NQ],
            )]

        def issue(j, sp, i):
            cps = piece_copy(j, sp, i)
            if len(cps) == NQ:
                @pl.when(t < NQ)
                def _():
                    for q in range(NQ):
                        @pl.when(t == q)
                        def _():
                            cps[q].start()
            else:
                @pl.when(t == 0)
                def _():
                    cps[0].start()

        def drain(j, sp, i):
            cps = piece_copy(j, sp, i)
            if len(cps) == NQ:
                for q in range(NQ):
                    @pl.when(t == q)
                    def _():
                        cps[q].wait()
            else:
                @pl.when(t == 0)
                def _():
                    cps[0].wait()

        @pl.loop(0, tasks_per_sc)
        def _task(p):
            tid = task0 + p
            j = tid // n_blk
            sp = tid % n_blk

            for i in range(NBUF):
                issue(j, sp, i)
            for i, (poff, pln) in enumerate(pieces):
                drain(j, sp, i)
                plsc.subcore_barrier()
                b = i % NBUF
                if pln == plen:
                    pltpu.sync_copy(bufs[b].at[e, :],
                                    row_v.at[pl.ds(poff, pln)])
                else:
                    pltpu.sync_copy(bufs[b].at[e, pl.ds(0, pln)],
                                    row_v.at[pl.ds(poff, pln)])
                plsc.subcore_barrier()
                if i + NBUF < n_pc:
                    issue(j, sp, i + NBUF)

            pltpu.sync_copy(tab_hbm.at[j, pl.ds(sp * 8, 8), pl.ds(V128, vtail)],
                            tail_v)
            for w in range(vtail // 16):
                row_v[pl.ds(V128 + w * 16, 16)] = tail_v[e, pl.ds(w * 16, 16)]

            for qq in range(4):
                pltpu.sync_copy(
                    cat_hbm.at[j, pl.ds(h * 64 + qq * 16, 16), :], cat_v
                )

                @pl.loop(0, 16)
                def _rows(a):
                    for bb in range(8):
                        ii = cat_v[a, pl.ds(bb * 16, 16)]
                        vals = plsc.load_gather(row_v, [ii])
                        out_v[pl.ds(a * 128 + bb * 16, 16)] = vals

                pltpu.sync_copy(
                    out_v, stage.at[e, pl.ds(h * half + qq * qchunk, qchunk)]
                )

            plsc.subcore_barrier()

            @pl.when(t == 15)
            def _flush():
                pltpu.sync_copy(stage, out_hbm.at[pl.ds(j * E + sp * 8, 8), :])
            plsc.subcore_barrier()

    return k


def kernel(x, tables):
    B, F = x.shape
    n_cat, V, E = tables.shape
    n_num = F - n_cat

    tab_t = jnp.transpose(tables, (0, 2, 1))
    cat_js = x[:, n_num:].astype(jnp.int32).T.reshape(n_cat, B // 128, 128)

    k = _embed_kernel(B, n_cat, V, E)
    emb_t = k(tab_t, cat_js)
    return jnp.concatenate([x[:, :n_num], emb_t.T], axis=1)
